# baseline (speedup 1.0000x reference)
import functools

import jax
import jax.numpy as jnp
import numpy as np
from jax import lax
from jax.experimental import pallas as pl
from jax.experimental.pallas import tpu as pltpu
from jax.experimental.pallas import tpu_sc as plsc

N = 10000
D = 128
E = 320000
BLK = 1000
NW = 32
EPT = E // NW
CH = 64


def _pre_body(u_ref, i_ref, wsr_ref, bsr_ref, wdr_ref, bdr_ref,
              wsq_ref, bsq_ref, wdq_ref, bdq_ref,
              wsf_ref, bsf_ref, wdf_ref, bdf_ref,
              fs_r, fd_r, fs_q, fd_q, fs_f, fd_f):
    u = u_ref[...]
    it = i_ref[...]

    def mm(x, w_ref, b_ref):
        return lax.dot_general(x, w_ref[...], (((1,), (1,)), ((), ())),
                               preferred_element_type=jnp.float32) + b_ref[...]

    fs_r[...] = mm(u, wsr_ref, bsr_ref)
    fd_r[...] = mm(it, wdr_ref, bdr_ref)
    fs_q[...] = mm(it, wsq_ref, bsq_ref)
    fd_q[...] = mm(u, wdq_ref, bdq_ref)
    fs_f[...] = mm(u, wsf_ref, bsf_ref)
    fd_f[...] = mm(u, wdf_ref, bdf_ref)


def _pre(user_feat, item_feat, Ws_r, bs_r, Wd_r, bd_r,
         Ws_q, bs_q, Wd_q, bd_q, Ws_f, bs_f, Wd_f, bd_f):
    row = pl.BlockSpec((BLK, D), lambda i: (i, 0))
    full = pl.BlockSpec((D, D), lambda i: (0, 0))
    vec = pl.BlockSpec((D,), lambda i: (0,))
    return pl.pallas_call(
        _pre_body,
        grid=(N // BLK,),
        in_specs=[row, row] + [full, vec] * 6,
        out_specs=[row] * 6,
        out_shape=[jax.ShapeDtypeStruct((N, D), jnp.float32)] * 6,
    )(user_feat, item_feat, Ws_r, bs_r, Wd_r, bd_r,
      Ws_q, bs_q, Wd_q, bd_q, Ws_f, bs_f, Wd_f, bd_f)


_GDN = lax.GatherDimensionNumbers(offset_dims=(), collapsed_slice_dims=(0,),
                                  start_index_map=(0,))


def _lane_perm(v, p):
    return lax.gather(v, p.reshape(16, 1), dimension_numbers=_GDN,
                      slice_sizes=(1,), mode=lax.GatherScatterMode.PROMISE_IN_BOUNDS)


def _sum_bcast(v, lanes):
    for m in (1, 2, 4, 8):
        v = v + _lane_perm(v, lanes ^ m)
    return v


def _edge_body(fs_hbm, fd_hbm, src_hbm, dst_hbm, attn_hbm,
               num_out, den_out,
               blk_s, blk_d, idx_d, idx_h, idx_dt, idx_ht,
               el0, el1, er0, er1, exw, attn_v, sh_num, sh_den,
               gs0, gs1, gs2, gs3):
    c = lax.axis_index("c")
    s = lax.axis_index("s")

    def for_region(fn):
        @pl.when(s < 15)
        def _main():
            base = s * 632
            for k in range(9):
                fn(base + k * 64, 64)
            fn(base + 576, 56)

        @pl.when(s == 15)
        def _tail():
            for k in range(8):
                fn(9480 + k * 64, 64)
            fn(9992, 8)

    def for_region_den(fn):
        @pl.when(s < 15)
        def _main():
            fn(s * 40, 40)

        @pl.when(s == 15)
        def _tail():
            fn(600, 25)

    zero16 = jnp.zeros((16,), jnp.float32)

    def zrow(r, _):
        for j in range(8):
            er0[r, pl.ds(j * 16, 16)] = zero16
        return 0

    lax.fori_loop(0, CH, zrow, 0, unroll=False)

    def zcopy(off, n):
        pltpu.sync_copy(er0.at[pl.ds(0, n)], sh_num.at[pl.ds(off, n)])

    def zcopy_den(off, n):
        pltpu.sync_copy(er0.at[pl.ds(0, n)], sh_den.at[pl.ds(off, n)])

    for_region(zcopy)
    for_region_den(zcopy_den)
    pltpu.sync_copy(attn_hbm, attn_v)
    plsc.subcore_barrier()

    attn_w = tuple(attn_v[pl.ds(j * 16, 16)] for j in range(8))
    lanes = lax.iota(jnp.int32, 16)
    lanes_hi = lax.shift_right_logical(lanes, 3)
    wid_base = (c * 16 + s) * EPT

    def do_group(elb, erb, base16, inb_base):
        def ebody(e, _):
            acc = jnp.zeros((16,), jnp.float32)
            avs = []
            for j in range(8):
                a = elb[base16 + e, pl.ds(j * 16, 16)]
                b = erb[base16 + e, pl.ds(j * 16, 16)]
                avs.append(a)
                t = a + b
                t = jnp.maximum(t, 0.2 * t)
                acc = acc + t * attn_w[j]
            ex = jnp.exp(_sum_bcast(acc, lanes))
            d_full = blk_d[pl.ds(inb_base + e, 16)][0]
            grp = ((d_full & 15) >> 1) * 16
            half = d_full & 1
            for j in range(8):
                exw[base16 + e, pl.ds(j * 16, 16)] = zero16
            exw[base16 + e, pl.ds(grp, 16)] = jnp.where(lanes_hi == half, ex, 0.0)
            for j in range(8):
                erb[base16 + e, pl.ds(j * 16, 16)] = avs[j] * ex
            return 0

        lax.fori_loop(0, 16, ebody, 0, unroll=2)

    def do_chunk(elb, erb, inb):
        for g in range(CH // 16):
            idx16 = blk_d[pl.ds(inb + g * 16, 16)]
            idx_d[pl.ds(g * 16, 16)] = idx16
            idx_h[pl.ds(g * 16, 16)] = lax.shift_right_logical(idx16, 4)
            do_group(elb, erb, g * 16, inb + g * 16)
        pltpu.sync_copy(exw, sh_den.at[idx_h], add=True)
        pltpu.sync_copy(erb, sh_num.at[idx_d], add=True)

    def pair_body(p, _):
        pm3 = lax.rem(p, 3)
        inb = pm3 * (2 * CH)

        @pl.when(pm3 == 0)
        def _refill():
            boff = wid_base + lax.div(p, 3) * 384
            pltpu.sync_copy(src_hbm.at[pl.ds(boff, 384)], blk_s)
            pltpu.sync_copy(dst_hbm.at[pl.ds(boff, 384)], blk_d)
            pltpu.async_copy(fs_hbm.at[blk_s.at[pl.ds(inb, CH)]], el0, gs0)
            pltpu.async_copy(fd_hbm.at[blk_d.at[pl.ds(inb, CH)]], er0, gs1)

        cpb1 = pltpu.async_copy(fs_hbm.at[blk_s.at[pl.ds(inb + CH, CH)]], el1, gs2)
        cpb2 = pltpu.async_copy(fd_hbm.at[blk_d.at[pl.ds(inb + CH, CH)]], er1, gs3)
        pltpu.make_async_copy(fs_hbm.at[blk_s.at[pl.ds(inb, CH)]], el0, gs0).wait()
        pltpu.make_async_copy(fd_hbm.at[blk_d.at[pl.ds(inb, CH)]], er0, gs1).wait()
        do_chunk(el0, er0, inb)

        @pl.when(pm3 != 2)
        def _prefetch():
            pltpu.async_copy(fs_hbm.at[blk_s.at[pl.ds(inb + 2 * CH, CH)]], el0, gs0)
            pltpu.async_copy(fd_hbm.at[blk_d.at[pl.ds(inb + 2 * CH, CH)]], er0, gs1)

        cpb1.wait()
        cpb2.wait()
        do_chunk(el1, er1, inb + CH)
        return 0

    lax.fori_loop(0, 78, pair_body, 0, unroll=False)

    toff = wid_base + 9984
    pltpu.sync_copy(src_hbm.at[pl.ds(toff, 16)], blk_s.at[pl.ds(0, 16)])
    pltpu.sync_copy(dst_hbm.at[pl.ds(toff, 16)], blk_d.at[pl.ds(0, 16)])
    cpt1 = pltpu.async_copy(fs_hbm.at[blk_s.at[pl.ds(0, 16)]], el0.at[pl.ds(0, 16)], gs0)
    cpt2 = pltpu.async_copy(fd_hbm.at[blk_d.at[pl.ds(0, 16)]], er0.at[pl.ds(0, 16)], gs1)
    cpt1.wait()
    cpt2.wait()
    idx16t = blk_d[pl.ds(0, 16)]
    idx_dt[pl.ds(0, 16)] = idx16t
    idx_ht[pl.ds(0, 16)] = lax.shift_right_logical(idx16t, 4)
    do_group(el0, er0, 0, 0)
    pltpu.sync_copy(exw.at[pl.ds(0, 16)], sh_den.at[idx_ht], add=True)
    pltpu.sync_copy(er0.at[pl.ds(0, 16)], sh_num.at[idx_dt], add=True)

    plsc.subcore_barrier()

    def dcopy(off, n):
        pltpu.sync_copy(sh_num.at[pl.ds(off, n)], el0.at[pl.ds(0, n)])
        pltpu.sync_copy(el0.at[pl.ds(0, n)], num_out.at[c, pl.ds(off, n)])

    def dcopy_den(off, n):
        pltpu.sync_copy(sh_den.at[pl.ds(off, n)], el0.at[pl.ds(0, n)])
        pltpu.sync_copy(el0.at[pl.ds(0, n)], den_out.at[c, pl.ds(off, n)])

    for_region(dcopy)
    for_region_den(dcopy_den)


@functools.partial(jax.jit, static_argnames=())
def _edge_pass(fs, fd, src, dst, attn):
    mesh = plsc.VectorSubcoreMesh(core_axis_name="c", subcore_axis_name="s")
    f = pl.kernel(
        _edge_body,
        out_type=[jax.ShapeDtypeStruct((2, N, D), jnp.float32),
                  jax.ShapeDtypeStruct((2, N // 16, D), jnp.float32)],
        mesh=mesh,
        scratch_types=[
            pltpu.VMEM((384,), jnp.int32),
            pltpu.VMEM((384,), jnp.int32),
            pltpu.VMEM((CH,), jnp.int32),
            pltpu.VMEM((CH,), jnp.int32),
            pltpu.VMEM((16,), jnp.int32),
            pltpu.VMEM((16,), jnp.int32),
            pltpu.VMEM((CH, D), jnp.float32),
            pltpu.VMEM((CH, D), jnp.float32),
            pltpu.VMEM((CH, D), jnp.float32),
            pltpu.VMEM((CH, D), jnp.float32),
            pltpu.VMEM((CH, D), jnp.float32),
            pltpu.VMEM((D,), jnp.float32),
            pltpu.VMEM_SHARED((N, D), jnp.float32),
            pltpu.VMEM_SHARED((N // 16, D), jnp.float32),
            pltpu.SemaphoreType.DMA,
            pltpu.SemaphoreType.DMA,
            pltpu.SemaphoreType.DMA,
            pltpu.SemaphoreType.DMA,
        ],
    )
    return f(fs, fd, src, dst, attn)


def _post_body(u_ref, i_ref, nr_ref, dr_ref, nq_ref, dq_ref, nf_ref, df_ref,
               w1i_ref, b1i_ref, w2i_ref, b2i_ref,
               w1t_ref, b1t_ref, w2t_ref, b2t_ref,
               uo_ref, io_ref):
    u = u_ref[...]
    it = i_ref[...]

    def safediv(n_ref, d_ref):
        num = n_ref[0] + n_ref[1]
        den = d_ref[0] + d_ref[1]
        ok = den > 0.0
        return jnp.where(ok, num / jnp.where(ok, den, 1.0), 0.0)

    io_ref[...] = safediv(nr_ref, dr_ref) + it
    p = safediv(nf_ref, df_ref)
    q = safediv(nq_ref, dq_ref)

    def score(x_a, x_b, w1_ref, b1_ref, w2_ref, b2_ref):
        w1 = w1_ref[...]
        h = lax.dot_general(x_a, w1[:, :D], (((1,), (1,)), ((), ())),
                            preferred_element_type=jnp.float32)
        h = h + lax.dot_general(x_b, w1[:, D:], (((1,), (1,)), ((), ())),
                                preferred_element_type=jnp.float32)
        h = h + b1_ref[...]
        sc = lax.dot_general(h, w2_ref[...], (((1,), (0,)), ((), ())),
                             preferred_element_type=jnp.float32) + b2_ref[0]
        return jnp.maximum(sc, 0.01 * sc)

    s_inf = score(u, p, w1i_ref, b1i_ref, w2i_ref, b2i_ref)
    s_int = score(u, q, w1t_ref, b1t_ref, w2t_ref, b2t_ref)
    m = jnp.maximum(s_inf, s_int)
    e0 = jnp.exp(s_inf - m)
    e1 = jnp.exp(s_int - m)
    uo_ref[...] = (e0 * p + e1 * q) / (e0 + e1) + u


def _post(user_feat, item_feat, numR, denR, numQ, denQ, numF, denF,
          W1_inf, b1_inf, W2_inf, b2_inf, W1_int, b1_int, W2_int, b2_int):
    row = pl.BlockSpec((BLK, D), lambda i: (i, 0))
    prow = pl.BlockSpec((2, BLK, D), lambda i: (0, i, 0))
    pden = pl.BlockSpec((2, BLK, 1), lambda i: (0, i, 0))
    w1 = pl.BlockSpec((D, 2 * D), lambda i: (0, 0))
    vec = pl.BlockSpec((D,), lambda i: (0,))
    w2 = pl.BlockSpec((D, 1), lambda i: (0, 0))
    b2 = pl.BlockSpec(memory_space=pltpu.SMEM)
    return pl.pallas_call(
        _post_body,
        grid=(N // BLK,),
        in_specs=[row, row, prow, pden, prow, pden, prow, pden,
                  w1, vec, w2, b2, w1, vec, w2, b2],
        out_specs=[row, row],
        out_shape=[jax.ShapeDtypeStruct((N, D), jnp.float32)] * 2,
    )(user_feat, item_feat, numR, denR, numQ, denQ, numF, denF,
      W1_inf, b1_inf, W2_inf, b2_inf, W1_int, b1_int, W2_int, b2_int)


def kernel(user_feat, item_feat, edge_rate, edge_rated_by, edge_follow,
           Wsrc_rate, bsrc_rate, Wdst_rate, bdst_rate, attn_rate,
           Wsrc_ratedby, bsrc_ratedby, Wdst_ratedby, bdst_ratedby, attn_ratedby,
           Wsrc_follow, bsrc_follow, Wdst_follow, bdst_follow, attn_follow,
           W1_inf, b1_inf, W2_inf, b2_inf, W1_int, b1_int, W2_int, b2_int):
    fs_r, fd_r, fs_q, fd_q, fs_f, fd_f = _pre(
        user_feat, item_feat,
        Wsrc_rate, bsrc_rate, Wdst_rate, bdst_rate,
        Wsrc_ratedby, bsrc_ratedby, Wdst_ratedby, bdst_ratedby,
        Wsrc_follow, bsrc_follow, Wdst_follow, bdst_follow)

    numR, denR = _edge_pass(fs_r, fd_r, edge_rate[0].astype(jnp.int32),
                            edge_rate[1].astype(jnp.int32), attn_rate)
    numQ, denQ = _edge_pass(fs_q, fd_q, edge_rated_by[0].astype(jnp.int32),
                            edge_rated_by[1].astype(jnp.int32), attn_ratedby)
    numF, denF = _edge_pass(fs_f, fd_f, edge_follow[0].astype(jnp.int32),
                            edge_follow[1].astype(jnp.int32), attn_follow)

    denR = denR.reshape(2, N, 8)[:, :, 0:1]
    denQ = denQ.reshape(2, N, 8)[:, :, 0:1]
    denF = denF.reshape(2, N, 8)[:, :, 0:1]

    user_out, item_out = _post(
        user_feat, item_feat,
        numR, denR, numQ, denQ,
        numF, denF,
        W1_inf, b1_inf, W2_inf.reshape(D, 1), b2_inf,
        W1_int, b1_int, W2_int.reshape(D, 1), b2_int)
    return user_out, item_out


# speedup vs baseline: 1.1077x; 1.1077x over previous
import functools

import jax
import jax.numpy as jnp
import numpy as np
from jax import lax
from jax.experimental import pallas as pl
from jax.experimental.pallas import tpu as pltpu
from jax.experimental.pallas import tpu_sc as plsc

N = 10000
D = 128
E = 320000
BLK = 1000
NW = 32
EPT = E // NW
CH = 64


def _pre_body(u_ref, i_ref, wsr_ref, bsr_ref, wdr_ref, bdr_ref,
              wsq_ref, bsq_ref, wdq_ref, bdq_ref,
              wsf_ref, bsf_ref, wdf_ref, bdf_ref,
              fs_r, fd_r, fs_q, fd_q, fs_f, fd_f):
    u = u_ref[...]
    it = i_ref[...]

    def mm(x, w_ref, b_ref):
        return lax.dot_general(x, w_ref[...], (((1,), (1,)), ((), ())),
                               preferred_element_type=jnp.float32) + b_ref[...]

    fs_r[...] = mm(u, wsr_ref, bsr_ref)
    fd_r[...] = mm(it, wdr_ref, bdr_ref)
    fs_q[...] = mm(it, wsq_ref, bsq_ref)
    fd_q[...] = mm(u, wdq_ref, bdq_ref)
    fs_f[...] = mm(u, wsf_ref, bsf_ref)
    fd_f[...] = mm(u, wdf_ref, bdf_ref)


def _pre(user_feat, item_feat, Ws_r, bs_r, Wd_r, bd_r,
         Ws_q, bs_q, Wd_q, bd_q, Ws_f, bs_f, Wd_f, bd_f):
    row = pl.BlockSpec((BLK, D), lambda i: (i, 0))
    full = pl.BlockSpec((D, D), lambda i: (0, 0))
    vec = pl.BlockSpec((D,), lambda i: (0,))
    return pl.pallas_call(
        _pre_body,
        grid=(N // BLK,),
        in_specs=[row, row] + [full, vec] * 6,
        out_specs=[row] * 6,
        out_shape=[jax.ShapeDtypeStruct((N, D), jnp.float32)] * 6,
    )(user_feat, item_feat, Ws_r, bs_r, Wd_r, bd_r,
      Ws_q, bs_q, Wd_q, bd_q, Ws_f, bs_f, Wd_f, bd_f)


_GDN = lax.GatherDimensionNumbers(offset_dims=(), collapsed_slice_dims=(0,),
                                  start_index_map=(0,))


def _lane_perm(v, p):
    return lax.gather(v, p.reshape(16, 1), dimension_numbers=_GDN,
                      slice_sizes=(1,), mode=lax.GatherScatterMode.PROMISE_IN_BOUNDS)


def _sum_bcast(v, lanes):
    for m in (1, 2, 4, 8):
        v = v + _lane_perm(v, lanes ^ m)
    return v


def _edge_body(fs_hbm, fd_hbm, src_hbm, dst_hbm, attn_hbm,
               num_out, den_out,
               blk_s, blk_d, idx_d, idx_dt, idx_io,
               el0, el1, er0, er1, den80, attn_v, sh_num, sh_den,
               gs0, gs1, gs2, gs3):
    c = lax.axis_index("c")
    s = lax.axis_index("s")

    def for_region(fn):
        @pl.when(s < 15)
        def _main():
            base = s * 632
            for k in range(9):
                fn(base + k * 64, 64)
            fn(base + 576, 56)

        @pl.when(s == 15)
        def _tail():
            for k in range(8):
                fn(9480 + k * 64, 64)
            fn(9992, 8)

    def for_region_den(fn):
        @pl.when(s < 10)
        def _main():
            fn(s * 8, 8)

    zero16 = jnp.zeros((16,), jnp.float32)

    def zrow(r, _):
        for j in range(8):
            er0[r, pl.ds(j * 16, 16)] = zero16
        return 0

    lax.fori_loop(0, CH, zrow, 0, unroll=False)

    def zden(r, _):
        for j in range(8):
            den80[r, pl.ds(j * 16, 16)] = zero16
        return 0

    lax.fori_loop(0, 80, zden, 0, unroll=False)

    def zcopy(off, n):
        pltpu.sync_copy(er0.at[pl.ds(0, n)], sh_num.at[pl.ds(off, n)])

    def zcopy_den(off, n):
        pltpu.sync_copy(er0.at[pl.ds(0, n)], sh_den.at[pl.ds(off, n)])

    for_region(zcopy)
    for_region_den(zcopy_den)
    pltpu.sync_copy(attn_hbm, attn_v)
    plsc.subcore_barrier()

    attn_w = tuple(attn_v[pl.ds(j * 16, 16)] for j in range(8))
    lanes = lax.iota(jnp.int32, 16)
    lanes_hi = lax.shift_right_logical(lanes, 3)
    wid_base = (c * 16 + s) * EPT

    def do_group(elb, erb, base16, inb_base):
        def ebody(e, _):
            acc = jnp.zeros((16,), jnp.float32)
            avs = []
            for j in range(8):
                a = elb[base16 + e, pl.ds(j * 16, 16)]
                b = erb[base16 + e, pl.ds(j * 16, 16)]
                avs.append(a)
                t = a + b
                t = jnp.maximum(t, 0.2 * t)
                acc = acc + t * attn_w[j]
            ex = jnp.exp(_sum_bcast(acc, lanes))
            d_full = blk_d[pl.ds(inb_base + e, 16)][0]
            drow = lax.shift_right_logical(d_full, 7)
            dgrp = (lax.shift_right_logical(d_full, 4) & 7) * 16
            dpos = d_full & 15
            v = den80[drow, pl.ds(dgrp, 16)]
            den80[drow, pl.ds(dgrp, 16)] = v + jnp.where(lanes == dpos, ex, 0.0)
            for j in range(8):
                erb[base16 + e, pl.ds(j * 16, 16)] = avs[j] * ex
            return 0

        lax.fori_loop(0, 16, ebody, 0, unroll=False)

    def do_chunk(elb, erb, inb):
        for g in range(CH // 16):
            idx16 = blk_d[pl.ds(inb + g * 16, 16)]
            idx_d[pl.ds(g * 16, 16)] = idx16
            do_group(elb, erb, g * 16, inb + g * 16)
        pltpu.sync_copy(erb, sh_num.at[idx_d], add=True)

    def pair_body(p, _):
        pm3 = lax.rem(p, 3)
        inb = pm3 * (2 * CH)

        @pl.when(pm3 == 0)
        def _refill():
            boff = wid_base + lax.div(p, 3) * 384
            pltpu.sync_copy(src_hbm.at[pl.ds(boff, 384)], blk_s)
            pltpu.sync_copy(dst_hbm.at[pl.ds(boff, 384)], blk_d)
            pltpu.async_copy(fs_hbm.at[blk_s.at[pl.ds(inb, CH)]], el0, gs0)
            pltpu.async_copy(fd_hbm.at[blk_d.at[pl.ds(inb, CH)]], er0, gs1)

        cpb1 = pltpu.async_copy(fs_hbm.at[blk_s.at[pl.ds(inb + CH, CH)]], el1, gs2)
        cpb2 = pltpu.async_copy(fd_hbm.at[blk_d.at[pl.ds(inb + CH, CH)]], er1, gs3)
        pltpu.make_async_copy(fs_hbm.at[blk_s.at[pl.ds(inb, CH)]], el0, gs0).wait()
        pltpu.make_async_copy(fd_hbm.at[blk_d.at[pl.ds(inb, CH)]], er0, gs1).wait()
        do_chunk(el0, er0, inb)

        @pl.when(pm3 != 2)
        def _prefetch():
            pltpu.async_copy(fs_hbm.at[blk_s.at[pl.ds(inb + 2 * CH, CH)]], el0, gs0)
            pltpu.async_copy(fd_hbm.at[blk_d.at[pl.ds(inb + 2 * CH, CH)]], er0, gs1)

        cpb1.wait()
        cpb2.wait()
        do_chunk(el1, er1, inb + CH)
        return 0

    lax.fori_loop(0, 78, pair_body, 0, unroll=False)

    toff = wid_base + 9984
    pltpu.sync_copy(src_hbm.at[pl.ds(toff, 16)], blk_s.at[pl.ds(0, 16)])
    pltpu.sync_copy(dst_hbm.at[pl.ds(toff, 16)], blk_d.at[pl.ds(0, 16)])
    cpt1 = pltpu.async_copy(fs_hbm.at[blk_s.at[pl.ds(0, 16)]], el0.at[pl.ds(0, 16)], gs0)
    cpt2 = pltpu.async_copy(fd_hbm.at[blk_d.at[pl.ds(0, 16)]], er0.at[pl.ds(0, 16)], gs1)
    cpt1.wait()
    cpt2.wait()
    idx16t = blk_d[pl.ds(0, 16)]
    idx_dt[pl.ds(0, 16)] = idx16t
    do_group(el0, er0, 0, 0)
    pltpu.sync_copy(er0.at[pl.ds(0, 16)], sh_num.at[idx_dt], add=True)

    def zio(g, _):
        idx_io[pl.ds(g * 16, 16)] = lanes + g * 16
        return 0

    lax.fori_loop(0, 5, zio, 0, unroll=False)
    pltpu.sync_copy(den80, sh_den.at[idx_io], add=True)

    plsc.subcore_barrier()

    def dcopy(off, n):
        pltpu.sync_copy(sh_num.at[pl.ds(off, n)], el0.at[pl.ds(0, n)])
        pltpu.sync_copy(el0.at[pl.ds(0, n)], num_out.at[c, pl.ds(off, n)])

    def dcopy_den(off, n):
        pltpu.sync_copy(sh_den.at[pl.ds(off, n)], el0.at[pl.ds(0, n)])
        pltpu.sync_copy(el0.at[pl.ds(0, n)], den_out.at[c, pl.ds(off, n)])

    for_region(dcopy)
    for_region_den(dcopy_den)


@functools.partial(jax.jit, static_argnames=())
def _edge_pass(fs, fd, src, dst, attn):
    mesh = plsc.VectorSubcoreMesh(core_axis_name="c", subcore_axis_name="s")
    f = pl.kernel(
        _edge_body,
        out_type=[jax.ShapeDtypeStruct((2, N, D), jnp.float32),
                  jax.ShapeDtypeStruct((2, 80, D), jnp.float32)],
        mesh=mesh,
        scratch_types=[
            pltpu.VMEM((384,), jnp.int32),
            pltpu.VMEM((384,), jnp.int32),
            pltpu.VMEM((CH,), jnp.int32),
            pltpu.VMEM((16,), jnp.int32),
            pltpu.VMEM((80,), jnp.int32),
            pltpu.VMEM((CH, D), jnp.float32),
            pltpu.VMEM((CH, D), jnp.float32),
            pltpu.VMEM((CH, D), jnp.float32),
            pltpu.VMEM((CH, D), jnp.float32),
            pltpu.VMEM((80, D), jnp.float32),
            pltpu.VMEM((D,), jnp.float32),
            pltpu.VMEM_SHARED((N, D), jnp.float32),
            pltpu.VMEM_SHARED((80, D), jnp.float32),
            pltpu.SemaphoreType.DMA,
            pltpu.SemaphoreType.DMA,
            pltpu.SemaphoreType.DMA,
            pltpu.SemaphoreType.DMA,
        ],
    )
    return f(fs, fd, src, dst, attn)


def _post_body(u_ref, i_ref, nr_ref, dr_ref, nq_ref, dq_ref, nf_ref, df_ref,
               w1i_ref, b1i_ref, w2i_ref, b2i_ref,
               w1t_ref, b1t_ref, w2t_ref, b2t_ref,
               uo_ref, io_ref):
    u = u_ref[...]
    it = i_ref[...]

    def safediv(n_ref, d_ref):
        num = n_ref[0] + n_ref[1]
        den = d_ref[0] + d_ref[1]
        ok = den > 0.0
        return jnp.where(ok, num / jnp.where(ok, den, 1.0), 0.0)

    io_ref[...] = safediv(nr_ref, dr_ref) + it
    p = safediv(nf_ref, df_ref)
    q = safediv(nq_ref, dq_ref)

    def score(x_a, x_b, w1_ref, b1_ref, w2_ref, b2_ref):
        w1 = w1_ref[...]
        h = lax.dot_general(x_a, w1[:, :D], (((1,), (1,)), ((), ())),
                            preferred_element_type=jnp.float32)
        h = h + lax.dot_general(x_b, w1[:, D:], (((1,), (1,)), ((), ())),
                                preferred_element_type=jnp.float32)
        h = h + b1_ref[...]
        sc = lax.dot_general(h, w2_ref[...], (((1,), (0,)), ((), ())),
                             preferred_element_type=jnp.float32) + b2_ref[0]
        return jnp.maximum(sc, 0.01 * sc)

    s_inf = score(u, p, w1i_ref, b1i_ref, w2i_ref, b2i_ref)
    s_int = score(u, q, w1t_ref, b1t_ref, w2t_ref, b2t_ref)
    m = jnp.maximum(s_inf, s_int)
    e0 = jnp.exp(s_inf - m)
    e1 = jnp.exp(s_int - m)
    uo_ref[...] = (e0 * p + e1 * q) / (e0 + e1) + u


def _post(user_feat, item_feat, numR, denR, numQ, denQ, numF, denF,
          W1_inf, b1_inf, W2_inf, b2_inf, W1_int, b1_int, W2_int, b2_int):
    row = pl.BlockSpec((BLK, D), lambda i: (i, 0))
    prow = pl.BlockSpec((2, BLK, D), lambda i: (0, i, 0))
    pden = pl.BlockSpec((2, BLK, 1), lambda i: (0, i, 0))
    w1 = pl.BlockSpec((D, 2 * D), lambda i: (0, 0))
    vec = pl.BlockSpec((D,), lambda i: (0,))
    w2 = pl.BlockSpec((D, 1), lambda i: (0, 0))
    b2 = pl.BlockSpec(memory_space=pltpu.SMEM)
    return pl.pallas_call(
        _post_body,
        grid=(N // BLK,),
        in_specs=[row, row, prow, pden, prow, pden, prow, pden,
                  w1, vec, w2, b2, w1, vec, w2, b2],
        out_specs=[row, row],
        out_shape=[jax.ShapeDtypeStruct((N, D), jnp.float32)] * 2,
    )(user_feat, item_feat, numR, denR, numQ, denQ, numF, denF,
      W1_inf, b1_inf, W2_inf, b2_inf, W1_int, b1_int, W2_int, b2_int)


def kernel(user_feat, item_feat, edge_rate, edge_rated_by, edge_follow,
           Wsrc_rate, bsrc_rate, Wdst_rate, bdst_rate, attn_rate,
           Wsrc_ratedby, bsrc_ratedby, Wdst_ratedby, bdst_ratedby, attn_ratedby,
           Wsrc_follow, bsrc_follow, Wdst_follow, bdst_follow, attn_follow,
           W1_inf, b1_inf, W2_inf, b2_inf, W1_int, b1_int, W2_int, b2_int):
    fs_r, fd_r, fs_q, fd_q, fs_f, fd_f = _pre(
        user_feat, item_feat,
        Wsrc_rate, bsrc_rate, Wdst_rate, bdst_rate,
        Wsrc_ratedby, bsrc_ratedby, Wdst_ratedby, bdst_ratedby,
        Wsrc_follow, bsrc_follow, Wdst_follow, bdst_follow)

    numR, denR = _edge_pass(fs_r, fd_r, edge_rate[0].astype(jnp.int32),
                            edge_rate[1].astype(jnp.int32), attn_rate)
    numQ, denQ = _edge_pass(fs_q, fd_q, edge_rated_by[0].astype(jnp.int32),
                            edge_rated_by[1].astype(jnp.int32), attn_ratedby)
    numF, denF = _edge_pass(fs_f, fd_f, edge_follow[0].astype(jnp.int32),
                            edge_follow[1].astype(jnp.int32), attn_follow)

    denR = denR.reshape(2, 80 * D)[:, :N, None]
    denQ = denQ.reshape(2, 80 * D)[:, :N, None]
    denF = denF.reshape(2, 80 * D)[:, :N, None]

    user_out, item_out = _post(
        user_feat, item_feat,
        numR, denR, numQ, denQ,
        numF, denF,
        W1_inf, b1_inf, W2_inf.reshape(D, 1), b2_inf,
        W1_int, b1_int, W2_int.reshape(D, 1), b2_int)
    return user_out, item_out
